# Initial kernel scaffold; baseline (speedup 1.0000x reference)
import jax
import jax.numpy as jnp
from jax.experimental import pallas as pl


def kernel(x, edge_index, W1, att_src1, att_dst1, b1, W2, att_src2, att_dst2, b2):
    raise NotImplementedError("write your pallas kernel here")


# speedup vs baseline: 40.4997x; 40.4997x over previous
import functools

import jax
import jax.numpy as jnp
from jax import lax
from jax.experimental import pallas as pl
from jax.experimental.pallas import tpu as pltpu
from jax.experimental.pallas import tpu_sc as plsc

N = 10000
E = 320000
D = 128
H = 8
C = 16
HP = 16
NC = 2
NS = 16
NW = NC * NS
EW = E // NW
K = 80
NCH = EW // K
WF = 10
RF = 1000
ZR = 40
MB = 1000
NEG_SLOPE = 0.2
EPS = 1e-16

_SDS = jax.ShapeDtypeStruct


def _tc_embed_body(x_ref, w_ref, as_ref, ad_ref, h_ref, asrc_ref, adst_ref):
    x = x_ref[...]
    h = jnp.dot(x, w_ref[...], preferred_element_type=jnp.float32)
    h_ref[...] = h
    j = lax.broadcasted_iota(jnp.int32, (D, HP), 0)
    p = lax.broadcasted_iota(jnp.int32, (D, HP), 1)
    seg = (j // C == p).astype(jnp.float32)
    asrc_ref[...] = jnp.dot(h * as_ref[...], seg,
                            preferred_element_type=jnp.float32)
    adst_ref[...] = jnp.dot(h * ad_ref[...], seg,
                            preferred_element_type=jnp.float32)


def _tc_embed(x, w, a_s, a_d):
    grid = (N // MB,)
    return pl.pallas_call(
        _tc_embed_body,
        grid=grid,
        in_specs=[
            pl.BlockSpec((MB, D), lambda i: (i, 0)),
            pl.BlockSpec((D, D), lambda i: (0, 0)),
            pl.BlockSpec((1, D), lambda i: (0, 0)),
            pl.BlockSpec((1, D), lambda i: (0, 0)),
        ],
        out_specs=[
            pl.BlockSpec((MB, D), lambda i: (i, 0)),
            pl.BlockSpec((MB, HP), lambda i: (i, 0)),
            pl.BlockSpec((MB, HP), lambda i: (i, 0)),
        ],
        out_shape=[
            _SDS((N, D), jnp.float32),
            _SDS((N, HP), jnp.float32),
            _SDS((N, HP), jnp.float32),
        ],
    )(x, w, a_s, a_d)


def _tc_norm_embed_body(o0_ref, o1_ref, d0_ref, d1_ref, b_ref, w_ref,
                        as_ref, ad_ref, h_ref, asrc_ref, adst_ref, den_ref):
    den = d0_ref[...] + d1_ref[...]
    den_ref[...] = den
    p = lax.broadcasted_iota(jnp.int32, (HP, D), 0)
    j = lax.broadcasted_iota(jnp.int32, (HP, D), 1)
    segt = (p == j // C).astype(jnp.float32)
    dexp = jnp.dot(den, segt, preferred_element_type=jnp.float32) + EPS
    x1 = (o0_ref[...] + o1_ref[...]) / dexp + b_ref[...]
    h = jnp.dot(x1, w_ref[...], preferred_element_type=jnp.float32)
    h_ref[...] = h
    j2 = lax.broadcasted_iota(jnp.int32, (D, HP), 0)
    p2 = lax.broadcasted_iota(jnp.int32, (D, HP), 1)
    seg = (j2 // C == p2).astype(jnp.float32)
    asrc_ref[...] = jnp.dot(h * as_ref[...], seg,
                            preferred_element_type=jnp.float32)
    adst_ref[...] = jnp.dot(h * ad_ref[...], seg,
                            preferred_element_type=jnp.float32)


def _tc_norm_embed(o0, o1, d0, d1, b, w, a_s, a_d):
    grid = (N // MB,)
    return pl.pallas_call(
        _tc_norm_embed_body,
        grid=grid,
        in_specs=[
            pl.BlockSpec((MB, D), lambda i: (i, 0)),
            pl.BlockSpec((MB, D), lambda i: (i, 0)),
            pl.BlockSpec((MB, HP), lambda i: (i, 0)),
            pl.BlockSpec((MB, HP), lambda i: (i, 0)),
            pl.BlockSpec((1, D), lambda i: (0, 0)),
            pl.BlockSpec((D, D), lambda i: (0, 0)),
            pl.BlockSpec((1, D), lambda i: (0, 0)),
            pl.BlockSpec((1, D), lambda i: (0, 0)),
        ],
        out_specs=[
            pl.BlockSpec((MB, D), lambda i: (i, 0)),
            pl.BlockSpec((MB, HP), lambda i: (i, 0)),
            pl.BlockSpec((MB, HP), lambda i: (i, 0)),
            pl.BlockSpec((MB, HP), lambda i: (i, 0)),
        ],
        out_shape=[
            _SDS((N, D), jnp.float32),
            _SDS((N, HP), jnp.float32),
            _SDS((N, HP), jnp.float32),
            _SDS((N, HP), jnp.float32),
        ],
    )(o0, o1, d0, d1, b, w, a_s, a_d)


def _tc_norm_body(o0_ref, o1_ref, d0_ref, d1_ref, b_ref, x_ref, den_ref):
    den = d0_ref[...] + d1_ref[...]
    den_ref[...] = den
    p = lax.broadcasted_iota(jnp.int32, (HP, D), 0)
    j = lax.broadcasted_iota(jnp.int32, (HP, D), 1)
    segt = (p == j // C).astype(jnp.float32)
    dexp = jnp.dot(den, segt, preferred_element_type=jnp.float32) + EPS
    x_ref[...] = (o0_ref[...] + o1_ref[...]) / dexp + b_ref[...]


def _tc_norm(o0, o1, d0, d1, b):
    grid = (N // MB,)
    return pl.pallas_call(
        _tc_norm_body,
        grid=grid,
        in_specs=[
            pl.BlockSpec((MB, D), lambda i: (i, 0)),
            pl.BlockSpec((MB, D), lambda i: (i, 0)),
            pl.BlockSpec((MB, HP), lambda i: (i, 0)),
            pl.BlockSpec((MB, HP), lambda i: (i, 0)),
            pl.BlockSpec((1, D), lambda i: (0, 0)),
        ],
        out_specs=[
            pl.BlockSpec((MB, D), lambda i: (i, 0)),
            pl.BlockSpec((MB, HP), lambda i: (i, 0)),
        ],
        out_shape=[
            _SDS((N, D), jnp.float32),
            _SDS((N, HP), jnp.float32),
        ],
    )(o0, o1, d0, d1, b)


_MESH = plsc.VectorSubcoreMesh(core_axis_name="c", subcore_axis_name="s",
                               num_cores=NC, num_subcores=NS)


@functools.partial(
    pl.kernel,
    mesh=_MESH,
    compiler_params=pltpu.CompilerParams(use_tc_tiling_on_sc=False, internal_scratch_in_bytes=131072),
    out_type=(
        _SDS((E, HP), jnp.float32),
        _SDS((NC, N, HP), jnp.float32),
        _SDS((NC, N, D), jnp.float32),
    ),
    scratch_types=[
        pltpu.VMEM((K,), jnp.int32),
        pltpu.VMEM((K,), jnp.int32),
        pltpu.VMEM((K, HP), jnp.float32),
        pltpu.VMEM((K, HP), jnp.float32),
        pltpu.VMEM((K, D), jnp.float32),
        pltpu.VMEM((K, HP), jnp.float32),
        pltpu.VMEM((ZR, D), jnp.float32),
        pltpu.VMEM((ZR, HP), jnp.float32),
        pltpu.VMEM_SHARED((N, HP), jnp.float32),
        pltpu.VMEM_SHARED((N, D), jnp.float32),
        pltpu.SemaphoreType.DMA,
    ],
)
def _sc_edge(h_hbm, asrc_hbm, adst_hbm, src_hbm, dst_hbm,
             ex_hbm, denp_hbm, outp_hbm,
             srcv, dstv, arows, brows, hrows, exrows, zw, zn,
             den_sp, out_sp, sem):
    cid = lax.axis_index("c")
    sid = lax.axis_index("s")
    wid = sid * NC + cid

    zero16 = jnp.zeros((16,), jnp.float32)

    @pl.when(sid < WF)
    def _init():
        def _zw_fill(r, _):
            for cc in range(D // 16):
                zw[r, pl.ds(cc * 16, 16)] = zero16
            return _

        lax.fori_loop(0, ZR, _zw_fill, None)

        def _zn_fill(r, _):
            zn[r, :] = zero16
            return _

        lax.fori_loop(0, ZR, _zn_fill, None)

        def _zcopy(i, _):
            off = sid * RF + i * ZR
            pltpu.sync_copy(zw, out_sp.at[pl.ds(off, ZR)])
            pltpu.sync_copy(zn, den_sp.at[pl.ds(off, ZR)])
            return _

        lax.fori_loop(0, RF // ZR, _zcopy, None)

    plsc.subcore_barrier()

    base_e = wid * EW

    def _chunk(ci, _):
        off = base_e + ci * K
        pltpu.sync_copy(src_hbm.at[pl.ds(off, K)], srcv)
        pltpu.sync_copy(dst_hbm.at[pl.ds(off, K)], dstv)
        c1 = pltpu.async_copy(asrc_hbm.at[srcv], arows, sem)
        c2 = pltpu.async_copy(adst_hbm.at[dstv], brows, sem)
        c3 = pltpu.async_copy(h_hbm.at[srcv], hrows, sem)
        c1.wait()
        c2.wait()
        c3.wait()

        def _edge(k, _):
            e = arows[k, :] + brows[k, :]
            e = jnp.maximum(e, e * NEG_SLOPE)
            ex = jnp.exp(e)
            exrows[k, :] = ex
            for hh in range(H):
                splat = lax.gather(
                    ex, jnp.full((16, 1), hh, jnp.int32),
                    lax.GatherDimensionNumbers(
                        offset_dims=(), collapsed_slice_dims=(0,),
                        start_index_map=(0,)),
                    (1,), mode=lax.GatherScatterMode.PROMISE_IN_BOUNDS)
                hseg = hrows[k, pl.ds(hh * 16, 16)]
                hrows[k, pl.ds(hh * 16, 16)] = hseg * splat
            return _

        lax.fori_loop(0, K, _edge, None)
        pltpu.sync_copy(exrows, den_sp.at[dstv], add=True)
        pltpu.sync_copy(hrows, out_sp.at[dstv], add=True)
        pltpu.sync_copy(exrows, ex_hbm.at[pl.ds(off, K)])
        return _

    lax.fori_loop(0, NCH, _chunk, None)
    plsc.subcore_barrier()

    @pl.when(sid < WF)
    def _flush():
        row0 = sid * RF
        pltpu.sync_copy(den_sp.at[pl.ds(row0, RF)],
                        denp_hbm.at[cid, pl.ds(row0, RF)])
        pltpu.sync_copy(out_sp.at[pl.ds(row0, RF)],
                        outp_hbm.at[cid, pl.ds(row0, RF)])


@functools.partial(
    pl.kernel,
    mesh=_MESH,
    compiler_params=pltpu.CompilerParams(use_tc_tiling_on_sc=False, internal_scratch_in_bytes=131072),
    out_type=_SDS((E, HP), jnp.float32),
    scratch_types=[
        pltpu.VMEM((K,), jnp.int32),
        pltpu.VMEM((K, HP), jnp.float32),
        pltpu.VMEM((K, HP), jnp.float32),
        pltpu.SemaphoreType.DMA,
    ],
)
def _sc_alpha(ex_hbm, den_hbm, dst_hbm, alpha_hbm, dstv, exrows, drows, sem):
    cid = lax.axis_index("c")
    sid = lax.axis_index("s")
    wid = sid * NC + cid
    base_e = wid * EW

    def _chunk(ci, _):
        off = base_e + ci * K
        pltpu.sync_copy(dst_hbm.at[pl.ds(off, K)], dstv)
        pltpu.sync_copy(ex_hbm.at[pl.ds(off, K)], exrows)
        pltpu.async_copy(den_hbm.at[dstv], drows, sem).wait()

        def _edge(k, _):
            exrows[k, :] = exrows[k, :] / (drows[k, :] + EPS)
            return _

        lax.fori_loop(0, K, _edge, None)
        pltpu.sync_copy(exrows, alpha_hbm.at[pl.ds(off, K)])
        return _

    lax.fori_loop(0, NCH, _chunk, None)


@jax.jit
def kernel(x, edge_index, W1, att_src1, att_dst1, b1,
           W2, att_src2, att_dst2, b2):
    src = edge_index[0].astype(jnp.int32)
    dst = edge_index[1].astype(jnp.int32)
    as1 = att_src1.reshape(1, D)
    ad1 = att_dst1.reshape(1, D)
    as2 = att_src2.reshape(1, D)
    ad2 = att_dst2.reshape(1, D)

    h1, asrc1, adst1 = _tc_embed(x, W1, as1, ad1)
    ex1, denp1, outp1 = _sc_edge(h1, asrc1, adst1, src, dst)
    h2, asrc2, adst2, den1 = _tc_norm_embed(
        outp1[0], outp1[1], denp1[0], denp1[1], b1.reshape(1, D), W2,
        as2, ad2)
    alpha1 = _sc_alpha(ex1, den1, dst)

    ex2, denp2, outp2 = _sc_edge(h2, asrc2, adst2, src, dst)
    x2, den2 = _tc_norm(outp2[0], outp2[1], denp2[0], denp2[1],
                        b2.reshape(1, D))
    alpha2 = _sc_alpha(ex2, den2, dst)

    return x2, alpha1[:, :H], alpha2[:, :H]


# speedup vs baseline: 59.9896x; 1.4812x over previous
import functools

import jax
import jax.numpy as jnp
from jax import lax
from jax.experimental import pallas as pl
from jax.experimental.pallas import tpu as pltpu
from jax.experimental.pallas import tpu_sc as plsc

N = 10000
E = 320000
D = 128
H = 8
C = 16
HP = 16
NC = 2
NS = 16
NW = NC * NS
EW = E // NW
K = 80
NCH = EW // K
WF = 10
RF = 1000
ZR = 40
MB = 1000
NEG_SLOPE = 0.2
EPS = 1e-16

_SDS = jax.ShapeDtypeStruct


def _tc_embed_body(x_ref, w_ref, as_ref, ad_ref, h_ref, asrc_ref, adst_ref):
    x = x_ref[...]
    h = jnp.dot(x, w_ref[...], preferred_element_type=jnp.float32)
    h_ref[...] = h
    j = lax.broadcasted_iota(jnp.int32, (D, HP), 0)
    p = lax.broadcasted_iota(jnp.int32, (D, HP), 1)
    seg = (j // C == p).astype(jnp.float32)
    asrc_ref[...] = jnp.dot(h * as_ref[...], seg,
                            preferred_element_type=jnp.float32)
    adst_ref[...] = jnp.dot(h * ad_ref[...], seg,
                            preferred_element_type=jnp.float32)


def _tc_embed(x, w, a_s, a_d):
    grid = (N // MB,)
    return pl.pallas_call(
        _tc_embed_body,
        grid=grid,
        in_specs=[
            pl.BlockSpec((MB, D), lambda i: (i, 0)),
            pl.BlockSpec((D, D), lambda i: (0, 0)),
            pl.BlockSpec((1, D), lambda i: (0, 0)),
            pl.BlockSpec((1, D), lambda i: (0, 0)),
        ],
        out_specs=[
            pl.BlockSpec((MB, D), lambda i: (i, 0)),
            pl.BlockSpec((MB, HP), lambda i: (i, 0)),
            pl.BlockSpec((MB, HP), lambda i: (i, 0)),
        ],
        out_shape=[
            _SDS((N, D), jnp.float32),
            _SDS((N, HP), jnp.float32),
            _SDS((N, HP), jnp.float32),
        ],
    )(x, w, a_s, a_d)


def _tc_norm_embed_body(o0_ref, o1_ref, d0_ref, d1_ref, b_ref, w_ref,
                        as_ref, ad_ref, h_ref, asrc_ref, adst_ref, den_ref):
    den = d0_ref[...] + d1_ref[...]
    den_ref[...] = den
    p = lax.broadcasted_iota(jnp.int32, (HP, D), 0)
    j = lax.broadcasted_iota(jnp.int32, (HP, D), 1)
    segt = (p == j // C).astype(jnp.float32)
    dexp = jnp.dot(den, segt, preferred_element_type=jnp.float32) + EPS
    x1 = (o0_ref[...] + o1_ref[...]) / dexp + b_ref[...]
    h = jnp.dot(x1, w_ref[...], preferred_element_type=jnp.float32)
    h_ref[...] = h
    j2 = lax.broadcasted_iota(jnp.int32, (D, HP), 0)
    p2 = lax.broadcasted_iota(jnp.int32, (D, HP), 1)
    seg = (j2 // C == p2).astype(jnp.float32)
    asrc_ref[...] = jnp.dot(h * as_ref[...], seg,
                            preferred_element_type=jnp.float32)
    adst_ref[...] = jnp.dot(h * ad_ref[...], seg,
                            preferred_element_type=jnp.float32)


def _tc_norm_embed(o0, o1, d0, d1, b, w, a_s, a_d):
    grid = (N // MB,)
    return pl.pallas_call(
        _tc_norm_embed_body,
        grid=grid,
        in_specs=[
            pl.BlockSpec((MB, D), lambda i: (i, 0)),
            pl.BlockSpec((MB, D), lambda i: (i, 0)),
            pl.BlockSpec((MB, HP), lambda i: (i, 0)),
            pl.BlockSpec((MB, HP), lambda i: (i, 0)),
            pl.BlockSpec((1, D), lambda i: (0, 0)),
            pl.BlockSpec((D, D), lambda i: (0, 0)),
            pl.BlockSpec((1, D), lambda i: (0, 0)),
            pl.BlockSpec((1, D), lambda i: (0, 0)),
        ],
        out_specs=[
            pl.BlockSpec((MB, D), lambda i: (i, 0)),
            pl.BlockSpec((MB, HP), lambda i: (i, 0)),
            pl.BlockSpec((MB, HP), lambda i: (i, 0)),
            pl.BlockSpec((MB, HP), lambda i: (i, 0)),
        ],
        out_shape=[
            _SDS((N, D), jnp.float32),
            _SDS((N, HP), jnp.float32),
            _SDS((N, HP), jnp.float32),
            _SDS((N, HP), jnp.float32),
        ],
    )(o0, o1, d0, d1, b, w, a_s, a_d)


def _tc_norm_body(o0_ref, o1_ref, d0_ref, d1_ref, b_ref, x_ref, den_ref):
    den = d0_ref[...] + d1_ref[...]
    den_ref[...] = den
    p = lax.broadcasted_iota(jnp.int32, (HP, D), 0)
    j = lax.broadcasted_iota(jnp.int32, (HP, D), 1)
    segt = (p == j // C).astype(jnp.float32)
    dexp = jnp.dot(den, segt, preferred_element_type=jnp.float32) + EPS
    x_ref[...] = (o0_ref[...] + o1_ref[...]) / dexp + b_ref[...]


def _tc_norm(o0, o1, d0, d1, b):
    grid = (N // MB,)
    return pl.pallas_call(
        _tc_norm_body,
        grid=grid,
        in_specs=[
            pl.BlockSpec((MB, D), lambda i: (i, 0)),
            pl.BlockSpec((MB, D), lambda i: (i, 0)),
            pl.BlockSpec((MB, HP), lambda i: (i, 0)),
            pl.BlockSpec((MB, HP), lambda i: (i, 0)),
            pl.BlockSpec((1, D), lambda i: (0, 0)),
        ],
        out_specs=[
            pl.BlockSpec((MB, D), lambda i: (i, 0)),
            pl.BlockSpec((MB, HP), lambda i: (i, 0)),
        ],
        out_shape=[
            _SDS((N, D), jnp.float32),
            _SDS((N, HP), jnp.float32),
        ],
    )(o0, o1, d0, d1, b)


_MESH = plsc.VectorSubcoreMesh(core_axis_name="c", subcore_axis_name="s",
                               num_cores=NC, num_subcores=NS)


def _splat(vec, hh):
    return lax.gather(
        vec, jnp.full((16, 1), hh, jnp.int32),
        lax.GatherDimensionNumbers(
            offset_dims=(), collapsed_slice_dims=(0,), start_index_map=(0,)),
        (1,), mode=lax.GatherScatterMode.PROMISE_IN_BOUNDS)


def _make_sc_edge(fuse_alpha):
    out_type = [
        _SDS((E, HP), jnp.float32),
        _SDS((NC, N, HP), jnp.float32),
        _SDS((NC, N, D), jnp.float32),
    ]
    scratch = [
        pltpu.VMEM((2, K), jnp.int32),
        pltpu.VMEM((2, K), jnp.int32),
        pltpu.VMEM((2, K, HP), jnp.float32),
        pltpu.VMEM((2, K, HP), jnp.float32),
        pltpu.VMEM((2, K, D), jnp.float32),
        pltpu.VMEM((2, K, HP), jnp.float32),
        pltpu.VMEM((ZR, D), jnp.float32),
        pltpu.VMEM((ZR, HP), jnp.float32),
        pltpu.VMEM_SHARED((N, HP), jnp.float32),
        pltpu.VMEM_SHARED((N, D), jnp.float32),
        pltpu.SemaphoreType.DMA,
        pltpu.SemaphoreType.DMA,
    ]
    if fuse_alpha:
        out_type.append(_SDS((E, HP), jnp.float32))
        scratch.append(pltpu.VMEM((2, K, HP), jnp.float32))
        scratch.append(pltpu.VMEM((2, K, HP), jnp.float32))

    def body(*refs):
        if fuse_alpha:
            (h_hbm, asrc_hbm, adst_hbm, src_hbm, dst_hbm, exp_hbm, denp_prev,
             ex_hbm, denp_hbm, outp_hbm, ap_hbm,
             srcv, dstv, arows, brows, hrows, exrows, zw, zn,
             den_sp, out_sp, sem0, sem1, drows, perows) = refs
        else:
            (h_hbm, asrc_hbm, adst_hbm, src_hbm, dst_hbm,
             ex_hbm, denp_hbm, outp_hbm,
             srcv, dstv, arows, brows, hrows, exrows, zw, zn,
             den_sp, out_sp, sem0, sem1) = refs
        sems = (sem0, sem1)
        cid = lax.axis_index("c")
        sid = lax.axis_index("s")
        wid = sid * NC + cid
        zero16 = jnp.zeros((16,), jnp.float32)

        @pl.when(sid < WF)
        def _init():
            def _zw_fill(r, _):
                for cc in range(D // 16):
                    zw[r, pl.ds(cc * 16, 16)] = zero16
                return _

            lax.fori_loop(0, ZR, _zw_fill, None)

            def _zn_fill(r, _):
                zn[r, :] = zero16
                return _

            lax.fori_loop(0, ZR, _zn_fill, None)

            def _zcopy(i, _):
                off = sid * RF + i * ZR
                pltpu.sync_copy(zw, out_sp.at[pl.ds(off, ZR)])
                pltpu.sync_copy(zn, den_sp.at[pl.ds(off, ZR)])
                return _

            lax.fori_loop(0, RF // ZR, _zcopy, None)

        plsc.subcore_barrier()
        base_e = wid * EW

        def _fire(ci, b):
            off = base_e + ci * K
            pltpu.sync_copy(src_hbm.at[pl.ds(off, K)], srcv.at[b])
            pltpu.sync_copy(dst_hbm.at[pl.ds(off, K)], dstv.at[b])
            pltpu.async_copy(asrc_hbm.at[srcv.at[b]], arows.at[b], sems[b])
            pltpu.async_copy(adst_hbm.at[dstv.at[b]], brows.at[b], sems[b])
            pltpu.async_copy(h_hbm.at[srcv.at[b]], hrows.at[b], sems[b])
            if fuse_alpha:
                pltpu.async_copy(denp_prev.at[dstv.at[b]], drows.at[b],
                                 sems[b])
                pltpu.async_copy(exp_hbm.at[pl.ds(off, K)], perows.at[b],
                                 sems[b])

        def _drain(b):
            pltpu.make_async_copy(asrc_hbm.at[srcv.at[b]], arows.at[b],
                                  sems[b]).wait()
            pltpu.make_async_copy(adst_hbm.at[dstv.at[b]], brows.at[b],
                                  sems[b]).wait()
            pltpu.make_async_copy(h_hbm.at[srcv.at[b]], hrows.at[b],
                                  sems[b]).wait()
            if fuse_alpha:
                pltpu.make_async_copy(denp_prev.at[dstv.at[b]], drows.at[b],
                                      sems[b]).wait()
                pltpu.make_async_copy(exp_hbm.at[pl.ds(0, K)], perows.at[b],
                                      sems[b]).wait()

        def _process(ci, b):
            off = base_e + ci * K

            def _edge(k, _):
                e = arows[b, k, :] + brows[b, k, :]
                e = jnp.maximum(e, e * NEG_SLOPE)
                ex = jnp.exp(e)
                exrows[b, k, :] = ex
                for hh in range(H):
                    hseg = hrows[b, k, pl.ds(hh * 16, 16)]
                    hrows[b, k, pl.ds(hh * 16, 16)] = hseg * _splat(ex, hh)
                if fuse_alpha:
                    perows[b, k, :] = perows[b, k, :] / (drows[b, k, :] + EPS)
                return _

            lax.fori_loop(0, K, _edge, None)
            pltpu.sync_copy(exrows.at[b], den_sp.at[dstv.at[b]], add=True)
            pltpu.sync_copy(hrows.at[b], out_sp.at[dstv.at[b]], add=True)
            pltpu.sync_copy(exrows.at[b], ex_hbm.at[pl.ds(off, K)])
            if fuse_alpha:
                pltpu.sync_copy(perows.at[b], ap_hbm.at[pl.ds(off, K)])

        _fire(0, 0)

        def _pair(g, _):
            for b in range(2):
                ci = 2 * g + b
                _drain(b)
                _fire(ci + 1, 1 - b)
                _process(ci, b)
            return _

        lax.fori_loop(0, (NCH - 1) // 2, _pair, None)
        _drain(0)
        _process(NCH - 1, 0)

        plsc.subcore_barrier()

        @pl.when(sid < WF)
        def _flush():
            row0 = sid * RF
            pltpu.sync_copy(den_sp.at[pl.ds(row0, RF)],
                            denp_hbm.at[cid, pl.ds(row0, RF)])
            pltpu.sync_copy(out_sp.at[pl.ds(row0, RF)],
                            outp_hbm.at[cid, pl.ds(row0, RF)])

    return pl.kernel(
        body,
        mesh=_MESH,
        compiler_params=pltpu.CompilerParams(use_tc_tiling_on_sc=False),
        out_type=tuple(out_type),
        scratch_types=scratch,
    )


_sc_edge = _make_sc_edge(False)
_sc_edge_fused = _make_sc_edge(True)


@functools.partial(
    pl.kernel,
    mesh=_MESH,
    compiler_params=pltpu.CompilerParams(use_tc_tiling_on_sc=False),
    out_type=_SDS((E, HP), jnp.float32),
    scratch_types=[
        pltpu.VMEM((2, K), jnp.int32),
        pltpu.VMEM((2, K, HP), jnp.float32),
        pltpu.VMEM((2, K, HP), jnp.float32),
        pltpu.SemaphoreType.DMA,
        pltpu.SemaphoreType.DMA,
    ],
)
def _sc_alpha(ex_hbm, den_hbm, dst_hbm, alpha_hbm, dstv, exrows, drows,
              sem0, sem1):
    sems = (sem0, sem1)
    cid = lax.axis_index("c")
    sid = lax.axis_index("s")
    wid = sid * NC + cid
    base_e = wid * EW

    def _fire(ci, b):
        off = base_e + ci * K
        pltpu.sync_copy(dst_hbm.at[pl.ds(off, K)], dstv.at[b])
        pltpu.async_copy(ex_hbm.at[pl.ds(off, K)], exrows.at[b], sems[b])
        pltpu.async_copy(den_hbm.at[dstv.at[b]], drows.at[b], sems[b])

    def _drain(b):
        pltpu.make_async_copy(ex_hbm.at[pl.ds(0, K)], exrows.at[b],
                              sems[b]).wait()
        pltpu.make_async_copy(den_hbm.at[dstv.at[b]], drows.at[b],
                              sems[b]).wait()

    def _process(ci, b):
        def _edge(k, _):
            exrows[b, k, :] = exrows[b, k, :] / (drows[b, k, :] + EPS)
            return _

        lax.fori_loop(0, K, _edge, None)
        pltpu.sync_copy(exrows.at[b], alpha_hbm.at[pl.ds(base_e + ci * K, K)])

    _fire(0, 0)

    def _pair(g, _):
        for b in range(2):
            ci = 2 * g + b
            _drain(b)
            _fire(ci + 1, 1 - b)
            _process(ci, b)
        return _

    lax.fori_loop(0, (NCH - 1) // 2, _pair, None)
    _drain(0)
    _process(NCH - 1, 0)


@jax.jit
def kernel(x, edge_index, W1, att_src1, att_dst1, b1,
           W2, att_src2, att_dst2, b2):
    src = edge_index[0].astype(jnp.int32)
    dst = edge_index[1].astype(jnp.int32)
    as1 = att_src1.reshape(1, D)
    ad1 = att_dst1.reshape(1, D)
    as2 = att_src2.reshape(1, D)
    ad2 = att_dst2.reshape(1, D)

    h1, asrc1, adst1 = _tc_embed(x, W1, as1, ad1)
    ex1, denp1, outp1 = _sc_edge(h1, asrc1, adst1, src, dst)
    h2, asrc2, adst2, den1 = _tc_norm_embed(
        outp1[0], outp1[1], denp1[0], denp1[1], b1.reshape(1, D), W2,
        as2, ad2)

    ex2, denp2, outp2, alpha1 = _sc_edge_fused(
        h2, asrc2, adst2, src, dst, ex1, den1)
    x2, den2 = _tc_norm(outp2[0], outp2[1], denp2[0], denp2[1],
                        b2.reshape(1, D))
    alpha2 = _sc_alpha(ex2, den2, dst)

    return x2, alpha1[:, :H], alpha2[:, :H]


# speedup vs baseline: 75.8669x; 1.2647x over previous
import functools

import jax
import jax.numpy as jnp
from jax import lax
from jax.experimental import pallas as pl
from jax.experimental.pallas import tpu as pltpu
from jax.experimental.pallas import tpu_sc as plsc

N = 10000
E = 320000
D = 128
H = 8
C = 16
HP = 16
NC = 2
NS = 16
NW = NC * NS
EW = E // NW
K = 80
NCH = EW // K
WF = 10
RF = 1000
ZR = 40
MB = 1000
NEG_SLOPE = 0.2
EPS = 1e-16

_SDS = jax.ShapeDtypeStruct


def _tc_embed_body(x_ref, w_ref, as_ref, ad_ref, h_ref, asrc_ref, adst_ref):
    x = x_ref[...]
    h = jnp.dot(x, w_ref[...], preferred_element_type=jnp.float32)
    h_ref[...] = h
    j = lax.broadcasted_iota(jnp.int32, (D, HP), 0)
    p = lax.broadcasted_iota(jnp.int32, (D, HP), 1)
    seg = (j // C == p).astype(jnp.float32)
    asrc_ref[...] = jnp.dot(h * as_ref[...], seg,
                            preferred_element_type=jnp.float32)
    adst_ref[...] = jnp.dot(h * ad_ref[...], seg,
                            preferred_element_type=jnp.float32)


def _tc_embed(x, w, a_s, a_d):
    grid = (N // MB,)
    return pl.pallas_call(
        _tc_embed_body,
        grid=grid,
        in_specs=[
            pl.BlockSpec((MB, D), lambda i: (i, 0)),
            pl.BlockSpec((D, D), lambda i: (0, 0)),
            pl.BlockSpec((1, D), lambda i: (0, 0)),
            pl.BlockSpec((1, D), lambda i: (0, 0)),
        ],
        out_specs=[
            pl.BlockSpec((MB, D), lambda i: (i, 0)),
            pl.BlockSpec((MB, HP), lambda i: (i, 0)),
            pl.BlockSpec((MB, HP), lambda i: (i, 0)),
        ],
        out_shape=[
            _SDS((N, D), jnp.float32),
            _SDS((N, HP), jnp.float32),
            _SDS((N, HP), jnp.float32),
        ],
    )(x, w, a_s, a_d)


def _tc_norm_embed_body(o0_ref, o1_ref, d0_ref, d1_ref, b_ref, w_ref,
                        as_ref, ad_ref, h_ref, asrc_ref, adst_ref, den_ref):
    den = d0_ref[...] + d1_ref[...]
    den_ref[...] = den
    p = lax.broadcasted_iota(jnp.int32, (HP, D), 0)
    j = lax.broadcasted_iota(jnp.int32, (HP, D), 1)
    segt = (p == j // C).astype(jnp.float32)
    dexp = jnp.dot(den, segt, preferred_element_type=jnp.float32) + EPS
    x1 = (o0_ref[...] + o1_ref[...]) / dexp + b_ref[...]
    h = jnp.dot(x1, w_ref[...], preferred_element_type=jnp.float32)
    h_ref[...] = h
    j2 = lax.broadcasted_iota(jnp.int32, (D, HP), 0)
    p2 = lax.broadcasted_iota(jnp.int32, (D, HP), 1)
    seg = (j2 // C == p2).astype(jnp.float32)
    asrc_ref[...] = jnp.dot(h * as_ref[...], seg,
                            preferred_element_type=jnp.float32)
    adst_ref[...] = jnp.dot(h * ad_ref[...], seg,
                            preferred_element_type=jnp.float32)


def _tc_norm_embed(o0, o1, d0, d1, b, w, a_s, a_d):
    grid = (N // MB,)
    return pl.pallas_call(
        _tc_norm_embed_body,
        grid=grid,
        in_specs=[
            pl.BlockSpec((MB, D), lambda i: (i, 0)),
            pl.BlockSpec((MB, D), lambda i: (i, 0)),
            pl.BlockSpec((MB, HP), lambda i: (i, 0)),
            pl.BlockSpec((MB, HP), lambda i: (i, 0)),
            pl.BlockSpec((1, D), lambda i: (0, 0)),
            pl.BlockSpec((D, D), lambda i: (0, 0)),
            pl.BlockSpec((1, D), lambda i: (0, 0)),
            pl.BlockSpec((1, D), lambda i: (0, 0)),
        ],
        out_specs=[
            pl.BlockSpec((MB, D), lambda i: (i, 0)),
            pl.BlockSpec((MB, HP), lambda i: (i, 0)),
            pl.BlockSpec((MB, HP), lambda i: (i, 0)),
            pl.BlockSpec((MB, HP), lambda i: (i, 0)),
        ],
        out_shape=[
            _SDS((N, D), jnp.float32),
            _SDS((N, HP), jnp.float32),
            _SDS((N, HP), jnp.float32),
            _SDS((N, HP), jnp.float32),
        ],
    )(o0, o1, d0, d1, b, w, a_s, a_d)


def _tc_norm_body(o0_ref, o1_ref, d0_ref, d1_ref, b_ref, x_ref, den_ref):
    den = d0_ref[...] + d1_ref[...]
    den_ref[...] = den
    p = lax.broadcasted_iota(jnp.int32, (HP, D), 0)
    j = lax.broadcasted_iota(jnp.int32, (HP, D), 1)
    segt = (p == j // C).astype(jnp.float32)
    dexp = jnp.dot(den, segt, preferred_element_type=jnp.float32) + EPS
    x_ref[...] = (o0_ref[...] + o1_ref[...]) / dexp + b_ref[...]


def _tc_norm(o0, o1, d0, d1, b):
    grid = (N // MB,)
    return pl.pallas_call(
        _tc_norm_body,
        grid=grid,
        in_specs=[
            pl.BlockSpec((MB, D), lambda i: (i, 0)),
            pl.BlockSpec((MB, D), lambda i: (i, 0)),
            pl.BlockSpec((MB, HP), lambda i: (i, 0)),
            pl.BlockSpec((MB, HP), lambda i: (i, 0)),
            pl.BlockSpec((1, D), lambda i: (0, 0)),
        ],
        out_specs=[
            pl.BlockSpec((MB, D), lambda i: (i, 0)),
            pl.BlockSpec((MB, HP), lambda i: (i, 0)),
        ],
        out_shape=[
            _SDS((N, D), jnp.float32),
            _SDS((N, HP), jnp.float32),
        ],
    )(o0, o1, d0, d1, b)


_MESH = plsc.VectorSubcoreMesh(core_axis_name="c", subcore_axis_name="s",
                               num_cores=NC, num_subcores=NS)


def _splat(vec, hh):
    return lax.gather(
        vec, jnp.full((16, 1), hh, jnp.int32),
        lax.GatherDimensionNumbers(
            offset_dims=(), collapsed_slice_dims=(0,), start_index_map=(0,)),
        (1,), mode=lax.GatherScatterMode.PROMISE_IN_BOUNDS)


def _make_sc_edge(fuse_alpha):
    out_type = [
        _SDS((E, HP), jnp.float32),
        _SDS((NC, N, HP), jnp.float32),
        _SDS((NC, N, D), jnp.float32),
    ]
    scratch = [
        pltpu.VMEM((2, K), jnp.int32),
        pltpu.VMEM((2, K), jnp.int32),
        pltpu.VMEM((2, K, HP), jnp.float32),
        pltpu.VMEM((2, K, HP), jnp.float32),
        pltpu.VMEM((2, K, D), jnp.float32),
        pltpu.VMEM((2, K, HP), jnp.float32),
        pltpu.VMEM((ZR, D), jnp.float32),
        pltpu.VMEM((ZR, HP), jnp.float32),
        pltpu.VMEM_SHARED((N, HP), jnp.float32),
        pltpu.VMEM_SHARED((N, D), jnp.float32),
        pltpu.SemaphoreType.DMA,
        pltpu.SemaphoreType.DMA,
    ]
    if fuse_alpha:
        out_type.append(_SDS((E, HP), jnp.float32))
        scratch.append(pltpu.VMEM((2, K, HP), jnp.float32))
        scratch.append(pltpu.VMEM((2, K, HP), jnp.float32))

    def body(*refs):
        if fuse_alpha:
            (h_hbm, asrc_hbm, adst_hbm, src_hbm, dst_hbm, exp_hbm, denp_prev,
             ex_hbm, denp_hbm, outp_hbm, ap_hbm,
             srcv, dstv, arows, brows, hrows, exrows, zw, zn,
             den_sp, out_sp, sem0, sem1, drows, perows) = refs
        else:
            (h_hbm, asrc_hbm, adst_hbm, src_hbm, dst_hbm,
             ex_hbm, denp_hbm, outp_hbm,
             srcv, dstv, arows, brows, hrows, exrows, zw, zn,
             den_sp, out_sp, sem0, sem1) = refs
        sems = (sem0, sem1)
        cid = lax.axis_index("c")
        sid = lax.axis_index("s")
        wid = sid * NC + cid
        zero16 = jnp.zeros((16,), jnp.float32)

        @pl.when(sid < WF)
        def _init():
            def _zw_fill(r, _):
                for cc in range(D // 16):
                    zw[r, pl.ds(cc * 16, 16)] = zero16
                return _

            lax.fori_loop(0, ZR, _zw_fill, None)

            def _zn_fill(r, _):
                zn[r, :] = zero16
                return _

            lax.fori_loop(0, ZR, _zn_fill, None)

            def _zcopy(i, _):
                off = sid * RF + i * ZR
                pltpu.sync_copy(zw, out_sp.at[pl.ds(off, ZR)])
                pltpu.sync_copy(zn, den_sp.at[pl.ds(off, ZR)])
                return _

            lax.fori_loop(0, RF // ZR, _zcopy, None)

        plsc.subcore_barrier()
        base_e = wid * EW

        def _fire(ci, b):
            off = base_e + ci * K
            pltpu.sync_copy(src_hbm.at[pl.ds(off, K)], srcv.at[b])
            pltpu.sync_copy(dst_hbm.at[pl.ds(off, K)], dstv.at[b])
            pltpu.async_copy(asrc_hbm.at[srcv.at[b]], arows.at[b], sems[b])
            pltpu.async_copy(adst_hbm.at[dstv.at[b]], brows.at[b], sems[b])
            pltpu.async_copy(h_hbm.at[srcv.at[b]], hrows.at[b], sems[b])
            if fuse_alpha:
                pltpu.async_copy(denp_prev.at[dstv.at[b]], drows.at[b],
                                 sems[b])
                pltpu.async_copy(exp_hbm.at[pl.ds(off, K)], perows.at[b],
                                 sems[b])

        def _drain(b):
            pltpu.make_async_copy(asrc_hbm.at[srcv.at[b]], arows.at[b],
                                  sems[b]).wait()
            pltpu.make_async_copy(adst_hbm.at[dstv.at[b]], brows.at[b],
                                  sems[b]).wait()
            pltpu.make_async_copy(h_hbm.at[srcv.at[b]], hrows.at[b],
                                  sems[b]).wait()
            if fuse_alpha:
                pltpu.make_async_copy(denp_prev.at[dstv.at[b]], drows.at[b],
                                      sems[b]).wait()
                pltpu.make_async_copy(exp_hbm.at[pl.ds(0, K)], perows.at[b],
                                      sems[b]).wait()

        def _process(ci, b):
            off = base_e + ci * K

            @plsc.parallel_loop(0, K, step=1, unroll=4)
            def _edge(k):
                e = arows[b, k, :] + brows[b, k, :]
                e = jnp.maximum(e, e * NEG_SLOPE)
                ex = jnp.exp(e)
                exrows[b, k, :] = ex
                for hh in range(H):
                    hseg = hrows[b, k, pl.ds(hh * 16, 16)]
                    hrows[b, k, pl.ds(hh * 16, 16)] = hseg * _splat(ex, hh)
                if fuse_alpha:
                    perows[b, k, :] = perows[b, k, :] / (drows[b, k, :] + EPS)
            pltpu.sync_copy(exrows.at[b], den_sp.at[dstv.at[b]], add=True)
            pltpu.sync_copy(hrows.at[b], out_sp.at[dstv.at[b]], add=True)
            pltpu.sync_copy(exrows.at[b], ex_hbm.at[pl.ds(off, K)])
            if fuse_alpha:
                pltpu.sync_copy(perows.at[b], ap_hbm.at[pl.ds(off, K)])

        _fire(0, 0)

        def _pair(g, _):
            for b in range(2):
                ci = 2 * g + b
                _drain(b)
                _fire(ci + 1, 1 - b)
                _process(ci, b)
            return _

        lax.fori_loop(0, (NCH - 1) // 2, _pair, None)
        _drain(0)
        _process(NCH - 1, 0)

        plsc.subcore_barrier()

        @pl.when(sid < WF)
        def _flush():
            row0 = sid * RF
            pltpu.sync_copy(den_sp.at[pl.ds(row0, RF)],
                            denp_hbm.at[cid, pl.ds(row0, RF)])
            pltpu.sync_copy(out_sp.at[pl.ds(row0, RF)],
                            outp_hbm.at[cid, pl.ds(row0, RF)])

    return pl.kernel(
        body,
        mesh=_MESH,
        compiler_params=pltpu.CompilerParams(use_tc_tiling_on_sc=False),
        out_type=tuple(out_type),
        scratch_types=scratch,
    )


_sc_edge = _make_sc_edge(False)
_sc_edge_fused = _make_sc_edge(True)


@functools.partial(
    pl.kernel,
    mesh=_MESH,
    compiler_params=pltpu.CompilerParams(use_tc_tiling_on_sc=False),
    out_type=_SDS((E, HP), jnp.float32),
    scratch_types=[
        pltpu.VMEM((2, K), jnp.int32),
        pltpu.VMEM((2, K, HP), jnp.float32),
        pltpu.VMEM((2, K, HP), jnp.float32),
        pltpu.SemaphoreType.DMA,
        pltpu.SemaphoreType.DMA,
    ],
)
def _sc_alpha(ex_hbm, den_hbm, dst_hbm, alpha_hbm, dstv, exrows, drows,
              sem0, sem1):
    sems = (sem0, sem1)
    cid = lax.axis_index("c")
    sid = lax.axis_index("s")
    wid = sid * NC + cid
    base_e = wid * EW

    def _fire(ci, b):
        off = base_e + ci * K
        pltpu.sync_copy(dst_hbm.at[pl.ds(off, K)], dstv.at[b])
        pltpu.async_copy(ex_hbm.at[pl.ds(off, K)], exrows.at[b], sems[b])
        pltpu.async_copy(den_hbm.at[dstv.at[b]], drows.at[b], sems[b])

    def _drain(b):
        pltpu.make_async_copy(ex_hbm.at[pl.ds(0, K)], exrows.at[b],
                              sems[b]).wait()
        pltpu.make_async_copy(den_hbm.at[dstv.at[b]], drows.at[b],
                              sems[b]).wait()

    def _process(ci, b):
        @plsc.parallel_loop(0, K, step=1, unroll=4)
        def _edge(k):
            exrows[b, k, :] = exrows[b, k, :] / (drows[b, k, :] + EPS)

        pltpu.sync_copy(exrows.at[b], alpha_hbm.at[pl.ds(base_e + ci * K, K)])

    _fire(0, 0)

    def _pair(g, _):
        for b in range(2):
            ci = 2 * g + b
            _drain(b)
            _fire(ci + 1, 1 - b)
            _process(ci, b)
        return _

    lax.fori_loop(0, (NCH - 1) // 2, _pair, None)
    _drain(0)
    _process(NCH - 1, 0)


@jax.jit
def kernel(x, edge_index, W1, att_src1, att_dst1, b1,
           W2, att_src2, att_dst2, b2):
    src = edge_index[0].astype(jnp.int32)
    dst = edge_index[1].astype(jnp.int32)
    as1 = att_src1.reshape(1, D)
    ad1 = att_dst1.reshape(1, D)
    as2 = att_src2.reshape(1, D)
    ad2 = att_dst2.reshape(1, D)

    h1, asrc1, adst1 = _tc_embed(x, W1, as1, ad1)
    ex1, denp1, outp1 = _sc_edge(h1, asrc1, adst1, src, dst)
    h2, asrc2, adst2, den1 = _tc_norm_embed(
        outp1[0], outp1[1], denp1[0], denp1[1], b1.reshape(1, D), W2,
        as2, ad2)

    ex2, denp2, outp2, alpha1 = _sc_edge_fused(
        h2, asrc2, adst2, src, dst, ex1, den1)
    x2, den2 = _tc_norm(outp2[0], outp2[1], denp2[0], denp2[1],
                        b2.reshape(1, D))
    alpha2 = _sc_alpha(ex2, den2, dst)

    return x2, alpha1[:, :H], alpha2[:, :H]
